# Initial kernel scaffold; baseline (speedup 1.0000x reference)
import jax
import jax.numpy as jnp
from jax.experimental import pallas as pl


def kernel(x, edge_index, edge_attr, merge_W0, merge_b0, proj_W0, mlp1_W0, mlp1_b0, mlp2_W0, mlp2_b0, merge_W1, merge_b1, proj_W1, mlp1_W1, mlp1_b1, mlp2_W1, mlp2_b1):
    raise NotImplementedError("write your pallas kernel here")


# speedup vs baseline: 2.8838x; 2.8838x over previous
import functools
import math

import jax
import jax.numpy as jnp
from jax import lax
from jax.experimental import pallas as pl
from jax.experimental.pallas import tpu as pltpu
from jax.experimental.pallas import tpu_sc as plsc

D = 128
LANES = 16
CHUNKS = D // LANES
BK = 32
SB = 16


def _leaky(v):
    return jnp.where(v > 0, v, 0.2 * v)


def _node_prologue_body(h_ref, mW_ref, mb_ref, pw_ref, g_ref, b_ref, sm_ref):
    h = h_ref[...]
    mW = mW_ref[...]
    a = jnp.dot(h, mW[:D], preferred_element_type=jnp.float32)
    bm = jnp.dot(h, mW[D:2 * D], preferred_element_type=jnp.float32)
    g_ref[:, :D] = a
    g_ref[:, D:] = h
    b_ref[...] = bm
    zs = a + bm + mb_ref[...]
    s = jnp.sum(_leaky(zs) * pw_ref[...], axis=1, keepdims=True)
    alpha = jnp.exp(jnp.clip(s, -5.0, 5.0))
    sm_ref[...] = h * alpha


def _node_prologue(h, mW, mb_row, pw_row):
    n = h.shape[0]
    rn = 1000
    grid = (n // rn,)
    kin = mW.shape[0]
    return pl.pallas_call(
        _node_prologue_body,
        grid=grid,
        in_specs=[
            pl.BlockSpec((rn, D), lambda i: (i, 0)),
            pl.BlockSpec((kin, D), lambda i: (0, 0)),
            pl.BlockSpec((1, D), lambda i: (0, 0)),
            pl.BlockSpec((1, D), lambda i: (0, 0)),
        ],
        out_specs=[
            pl.BlockSpec((rn, 2 * D), lambda i: (i, 0)),
            pl.BlockSpec((rn, D), lambda i: (i, 0)),
            pl.BlockSpec((rn, D), lambda i: (i, 0)),
        ],
        out_shape=[
            jax.ShapeDtypeStruct((n, 2 * D), jnp.float32),
            jax.ShapeDtypeStruct((n, D), jnp.float32),
            jax.ShapeDtypeStruct((n, D), jnp.float32),
        ],
    )(h, mW, mb_row, pw_row)


def _ce_body(ea_ref, w_ref, mb_ref, ce_ref):
    ce_ref[...] = (
        jnp.dot(ea_ref[...], w_ref[...], preferred_element_type=jnp.float32)
        + mb_ref[...]
    )


def _edge_ce(ea_pad, mWe, mb_row):
    e_pad, f = ea_pad.shape
    re = math.gcd(4096, e_pad)
    grid = (e_pad // re,)
    return pl.pallas_call(
        _ce_body,
        grid=grid,
        in_specs=[
            pl.BlockSpec((re, f), lambda i: (i, 0)),
            pl.BlockSpec((f, D), lambda i: (0, 0)),
            pl.BlockSpec((1, D), lambda i: (0, 0)),
        ],
        out_specs=pl.BlockSpec((re, D), lambda i: (i, 0)),
        out_shape=jax.ShapeDtypeStruct((e_pad, D), jnp.float32),
    )(ea_pad, mWe, mb_row)


def _epi_pro_body(p_ref, sm_ref, w1_ref, b1_ref, w2_ref, b2_ref,
                  mW_ref, mb_ref, pw_ref, g_ref, b_ref, sm2_ref):
    agg = jnp.sum(p_ref[...], axis=0) + sm_ref[...]
    t = _leaky(
        jnp.dot(agg, w1_ref[...], preferred_element_type=jnp.float32)
        + b1_ref[...])
    h = _leaky(
        jnp.dot(t, w2_ref[...], preferred_element_type=jnp.float32)
        + b2_ref[...])
    mW = mW_ref[...]
    a = jnp.dot(h, mW[:D], preferred_element_type=jnp.float32)
    bm = jnp.dot(h, mW[D:2 * D], preferred_element_type=jnp.float32)
    g_ref[:, :D] = a
    g_ref[:, D:] = h
    b_ref[...] = bm
    zs = a + bm + mb_ref[...]
    sc = jnp.sum(_leaky(zs) * pw_ref[...], axis=1, keepdims=True)
    alpha = jnp.exp(jnp.clip(sc, -5.0, 5.0))
    sm2_ref[...] = h * alpha


def _epi_pro(partials, sm, w1, b1_row, w2, b2_row, mW_next, mb_row_next,
             pw_row_next):
    nc = partials.shape[0]
    n = sm.shape[0]
    rn = 1000
    grid = (n // rn,)
    kin = mW_next.shape[0]
    return pl.pallas_call(
        _epi_pro_body,
        grid=grid,
        in_specs=[
            pl.BlockSpec((nc, rn, D), lambda i: (0, i, 0)),
            pl.BlockSpec((rn, D), lambda i: (i, 0)),
            pl.BlockSpec((D, D), lambda i: (0, 0)),
            pl.BlockSpec((1, D), lambda i: (0, 0)),
            pl.BlockSpec((D, D), lambda i: (0, 0)),
            pl.BlockSpec((1, D), lambda i: (0, 0)),
            pl.BlockSpec((kin, D), lambda i: (0, 0)),
            pl.BlockSpec((1, D), lambda i: (0, 0)),
            pl.BlockSpec((1, D), lambda i: (0, 0)),
        ],
        out_specs=[
            pl.BlockSpec((rn, 2 * D), lambda i: (i, 0)),
            pl.BlockSpec((rn, D), lambda i: (i, 0)),
            pl.BlockSpec((rn, D), lambda i: (i, 0)),
        ],
        out_shape=[
            jax.ShapeDtypeStruct((n, 2 * D), jnp.float32),
            jax.ShapeDtypeStruct((n, D), jnp.float32),
            jax.ShapeDtypeStruct((n, D), jnp.float32),
        ],
    )(partials, sm, w1, b1_row, w2, b2_row, mW_next, mb_row_next,
      pw_row_next)


def _epilogue_body(p_ref, sm_ref, w1_ref, b1_ref, w2_ref, b2_ref,
                   o_ref, *, final_act):
    agg = jnp.sum(p_ref[...], axis=0) + sm_ref[...]
    t = _leaky(
        jnp.dot(agg, w1_ref[...], preferred_element_type=jnp.float32)
        + b1_ref[...]
    )
    o = jnp.dot(t, w2_ref[...], preferred_element_type=jnp.float32) + b2_ref[...]
    if final_act:
        o = _leaky(o)
    o_ref[...] = o


def _epilogue(partials, sm, w1, b1_row, w2, b2_row, final_act):
    nc = partials.shape[0]
    n = sm.shape[0]
    rn = 1000
    grid = (n // rn,)
    return pl.pallas_call(
        functools.partial(_epilogue_body, final_act=final_act),
        grid=grid,
        in_specs=[
            pl.BlockSpec((nc, rn, D), lambda i: (0, i, 0)),
            pl.BlockSpec((rn, D), lambda i: (i, 0)),
            pl.BlockSpec((D, D), lambda i: (0, 0)),
            pl.BlockSpec((1, D), lambda i: (0, 0)),
            pl.BlockSpec((D, D), lambda i: (0, 0)),
            pl.BlockSpec((1, D), lambda i: (0, 0)),
        ],
        out_specs=pl.BlockSpec((rn, D), lambda i: (i, 0)),
        out_shape=jax.ShapeDtypeStruct((n, D), jnp.float32),
    )(partials, sm, w1, b1_row, w2, b2_row)


_GDN = lax.GatherDimensionNumbers(
    offset_dims=(), collapsed_slice_dims=(0,), start_index_map=(0,))


def _xlane_sum(v):
    for k in (8, 4, 2, 1):
        idx = lax.iota(jnp.int32, LANES) ^ k
        perm = lax.gather(v, idx[:, None], dimension_numbers=_GDN,
                          slice_sizes=(1,),
                          mode=lax.GatherScatterMode.PROMISE_IN_BOUNDS)
        v = v + perm
    return v


def _edge_sc_build(n, e_pad, nc, ns):
    epw = e_pad // (nc * ns)
    nb = epw // BK
    nsb = nb // SB
    n_acc = ((n + 1 + ns * 8 - 1) // (ns * 8)) * (ns * 8)
    rows_per_tile = n_acc // ns
    assert epw % (BK * SB) == 0

    mesh = plsc.VectorSubcoreMesh(core_axis_name="c", subcore_axis_name="s")

    @functools.partial(
        pl.kernel,
        out_type=jax.ShapeDtypeStruct((nc * n_acc, D), jnp.float32),
        mesh=mesh,
        compiler_params=pltpu.CompilerParams(use_tc_tiling_on_sc=False),
        scratch_types=[
            pltpu.VMEM((D,), jnp.float32),
            pltpu.VMEM((SB, BK), jnp.int32),
            pltpu.VMEM((SB, BK), jnp.int32),
            pltpu.VMEM((BK, 2 * D), jnp.float32),
            pltpu.VMEM((BK, 2 * D), jnp.float32),
            pltpu.VMEM((BK, D), jnp.float32),
            pltpu.VMEM((BK, D), jnp.float32),
            pltpu.VMEM((BK, D), jnp.float32),
            pltpu.VMEM((BK, D), jnp.float32),
            pltpu.VMEM((BK, D), jnp.float32),
            pltpu.VMEM((BK, D), jnp.float32),
            pltpu.VMEM_SHARED((n_acc, D), jnp.float32),
            pltpu.SemaphoreType.DMA,
            pltpu.SemaphoreType.DMA,
            pltpu.SemaphoreType.DMA,
            pltpu.SemaphoreType.DMA,
        ],
    )
    def edge_kernel(g_hbm, b_hbm, ce_hbm, src_hbm, dst_hbm,
                    pw_hbm, zeros_hbm, out_hbm,
                    pw_v, srcsb_v, dstsb_v, g_v0, g_v1, b_v0, b_v1,
                    ce_v0, ce_v1, msg_v0, msg_v1, accum,
                    sem_g0, sem_g1, sem_s0, sem_s1):
        c = lax.axis_index("c")
        s = lax.axis_index("s")
        pltpu.sync_copy(pw_hbm, pw_v)
        zero16 = jnp.zeros((LANES,), jnp.float32)

        row0 = s * rows_per_tile
        pltpu.sync_copy(zeros_hbm.at[pl.ds(row0, rows_per_tile)],
                        accum.at[pl.ds(row0, rows_per_tile)])
        plsc.subcore_barrier()

        pwc = [pw_v[pl.ds(dd * LANES, LANES)] for dd in range(CHUNKS)]
        wrow = (c * ns + s) * (epw // BK)

        gsets = [(g_v0, b_v0, ce_v0, sem_g0), (g_v1, b_v1, ce_v1, sem_g1)]
        msets = [(msg_v0, sem_s0), (msg_v1, sem_s1)]

        def superblock(jsb, carry):
            @pl.when(jsb > 0)
            def _():
                pltpu.make_async_copy(
                    ce_hbm.at[pl.ds(0, BK)], msg_v0, sem_s0).wait()
                pltpu.make_async_copy(
                    ce_hbm.at[pl.ds(0, BK)], msg_v1, sem_s1).wait()

            rowbase = wrow + jsb * SB
            pltpu.sync_copy(src_hbm.at[pl.ds(rowbase, SB)], srcsb_v)
            pltpu.sync_copy(dst_hbm.at[pl.ds(rowbase, SB)], dstsb_v)

            def issue_gather(k):
                gv, bv, cv, sg = gsets[k % 2]
                pltpu.async_copy(g_hbm.at[srcsb_v.at[k]], gv, sg)
                pltpu.async_copy(b_hbm.at[dstsb_v.at[k]], bv, sg)
                pltpu.async_copy(
                    ce_hbm.at[pl.ds((rowbase + k) * BK, BK)], cv, sg)

            issue_gather(0)
            for k in range(SB):
                gv, bv, cv, sg = gsets[k % 2]
                mv, ss = msets[k % 2]
                if k + 1 < SB:
                    issue_gather(k + 1)
                pltpu.make_async_copy(g_hbm.at[srcsb_v.at[k]], gv, sg).wait()
                pltpu.make_async_copy(b_hbm.at[dstsb_v.at[k]], bv, sg).wait()
                pltpu.make_async_copy(ce_hbm.at[pl.ds(0, BK)], cv, sg).wait()
                if k >= 2:
                    pltpu.make_async_copy(
                        ce_hbm.at[pl.ds(0, BK)], mv, ss).wait()

                def edge(e, carry2):
                    sv = zero16
                    for dd in range(CHUNKS):
                        sl = pl.ds(dd * LANES, LANES)
                        z = gv[e, sl] + bv[e, sl] + cv[e, sl]
                        sv = sv + jnp.maximum(z, 0.2 * z) * pwc[dd]
                    av = jnp.exp(jnp.clip(_xlane_sum(sv), -5.0, 5.0))
                    for dd in range(CHUNKS):
                        mv[e, pl.ds(dd * LANES, LANES)] = (
                            gv[e, pl.ds(D + dd * LANES, LANES)] * av)
                    return carry2

                lax.fori_loop(0, BK, edge, 0, unroll=2)
                pltpu.async_copy(mv, accum.at[dstsb_v.at[k]], ss, add=True)
            return carry

        lax.fori_loop(0, nsb, superblock, 0)
        pltpu.make_async_copy(ce_hbm.at[pl.ds(0, BK)], msg_v0, sem_s0).wait()
        pltpu.make_async_copy(ce_hbm.at[pl.ds(0, BK)], msg_v1, sem_s1).wait()

        plsc.subcore_barrier()
        pltpu.sync_copy(accum.at[pl.ds(row0, rows_per_tile)],
                        out_hbm.at[pl.ds(c * n_acc + row0, rows_per_tile)])

    return edge_kernel, n_acc


def kernel(x, edge_index, edge_attr,
           merge_W0, merge_b0, proj_W0, mlp1_W0, mlp1_b0, mlp2_W0, mlp2_b0,
           merge_W1, merge_b1, proj_W1, mlp1_W1, mlp1_b1, mlp2_W1, mlp2_b1):
    n, d = x.shape
    e = edge_index.shape[1]
    f = edge_attr.shape[1]
    assert d == D

    info = plsc.get_sparse_core_info()
    nc, ns = info.num_cores, info.num_subcores

    unit = nc * ns * BK * SB
    e_pad = ((e + unit - 1) // unit) * unit
    pad = e_pad - e
    src = jnp.concatenate(
        [edge_index[0], jnp.zeros((pad,), jnp.int32)]).reshape(e_pad // BK, BK)
    dst = jnp.concatenate(
        [edge_index[1], jnp.full((pad,), n, jnp.int32)]).reshape(e_pad // BK, BK)
    ea_pad = jnp.concatenate(
        [edge_attr, jnp.zeros((pad, f), jnp.float32)], axis=0)

    edge_call, n_acc = _edge_sc_build(n, e_pad, nc, ns)
    zeros_acc = jnp.zeros((n_acc, D), jnp.float32)

    g, b_tab, sm = _node_prologue(
        x, merge_W0, merge_b0.reshape(1, D), proj_W0.reshape(1, D))
    ce0 = _edge_ce(ea_pad, merge_W0[2 * D:], merge_b0.reshape(1, D))
    p0 = edge_call(g, b_tab, ce0, src, dst, proj_W0[:, 0], zeros_acc)
    g2, b2_tab, sm2 = _epi_pro(
        p0.reshape(nc, n_acc, D), sm,
        mlp1_W0, mlp1_b0.reshape(1, D), mlp2_W0, mlp2_b0.reshape(1, D),
        merge_W1, merge_b1.reshape(1, D), proj_W1.reshape(1, D))
    ce1 = _edge_ce(ea_pad, merge_W1[2 * D:], merge_b1.reshape(1, D))
    p1 = edge_call(g2, b2_tab, ce1, src, dst, proj_W1[:, 0], zeros_acc)
    return _epilogue(p1.reshape(nc, n_acc, D), sm2,
                     mlp1_W1, mlp1_b1.reshape(1, D),
                     mlp2_W1, mlp2_b1.reshape(1, D),
                     final_act=False)
